# baseline (speedup 1.0000x reference)
import functools

import jax
import jax.numpy as jnp
from jax import lax
from jax.experimental import pallas as pl
from jax.experimental.pallas import tpu as pltpu
from jax.experimental.pallas import tpu_sc as plsc

NC = 2
NS = 16
NW = NC * NS
L = 16
C = 40


def _decode(x, src, dst):
    n, d = x.shape
    e = src.shape[0]
    half = e // 2
    per_w = half // NW
    nk = per_w // C

    mesh = plsc.VectorSubcoreMesh(
        core_axis_name="c", subcore_axis_name="s",
        num_cores=NC, num_subcores=NS)

    @functools.partial(
        pl.kernel,
        out_type=jax.ShapeDtypeStruct((half,), jnp.float32),
        mesh=mesh,
        scratch_types=[
            [[pltpu.VMEM((C,), jnp.int32) for _ in range(4)]
             for _ in range(4)],
            [pltpu.VMEM((C, d), jnp.int32) for _ in range(4)],
            [pltpu.VMEM((C, d), jnp.int32) for _ in range(4)],
            pltpu.VMEM((per_w + 2 * L,), jnp.float32),
            pltpu.VMEM((3 * L * 17,), jnp.float32),
            pltpu.VMEM_SHARED((n, d), jnp.int32),
            [pltpu.SemaphoreType.DMA for _ in range(4)],
            pltpu.SemaphoreType.DMA,
            pltpu.SemaphoreType.DMA,
        ],
        compiler_params=pltpu.CompilerParams(needs_layout_passes=False),
    )
    def decode(x_hbm, src_hbm, dst_hbm, out_hbm,
               ias, set0, set1, ob, stg, xs, sis, sr0, sr1):
        wid = lax.axis_index("s") * NC + lax.axis_index("c")
        sid = lax.axis_index("s")
        base0 = wid * per_w
        lane = lax.iota(jnp.int32, L)
        sets = (set0, set1)
        srs = (sr0, sr1)

        rps = (n // NS) // 8 * 8
        off = pl.multiple_of(sid * rps, 8)
        pltpu.sync_copy(x_hbm.at[pl.ds(off, rps)], xs.at[pl.ds(off, rps)])
        tail = n - rps * NS
        if tail:
            @pl.when(sid == 0)
            def _tail():
                pltpu.sync_copy(x_hbm.at[pl.ds(rps * NS, tail)],
                                xs.at[pl.ds(rps * NS, tail)])
        plsc.subcore_barrier()

        def fire_idx(k, p):
            o = base0 + k * C
            pltpu.async_copy(src_hbm.at[pl.ds(o, C)], ias[p][0], sis[p])
            pltpu.async_copy(dst_hbm.at[pl.ds(o, C)], ias[p][1], sis[p])
            pltpu.async_copy(src_hbm.at[pl.ds(o + half, C)], ias[p][2],
                             sis[p])
            pltpu.async_copy(dst_hbm.at[pl.ds(o + half, C)], ias[p][3],
                             sis[p])

        def drain_idx(p):
            for q in range(4):
                pltpu.make_async_copy(src_hbm.at[pl.ds(0, C)], ias[p][q],
                                      sis[p]).wait()

        def fire_rows(p, pi):
            for q in range(4):
                pltpu.async_copy(xs.at[ias[pi][q]], sets[p][q], srs[p])

        def drain_rows(p):
            for q in range(4):
                pltpu.make_async_copy(xs.at[ias[0][q]], sets[p][q],
                                      srs[p]).wait()

        def compute(k, p):
            rs1, rd1, rs2, rd2 = sets[p]

            def edge_partial(j):
                ps = []
                for c0 in range(d // (2 * L)):
                    sl = pl.ds(c0 * L, L)
                    for rs, rd in ((rs1, rd1), (rs2, rd2)):
                        prod = (plsc.bitcast(rs[j, sl], jnp.bfloat16)
                                * plsc.bitcast(rd[j, sl], jnp.bfloat16))
                        p0, p1 = plsc.unpack(
                            prod, format=plsc.PackFormat.INTERLEAVED)
                        ps.append(p0)
                        ps.append(p1)
                while len(ps) > 1:
                    ps = [a + b for a, b in zip(ps[::2], ps[1::2])]
                return ps[0]

            @plsc.parallel_loop(0, C)
            def _j1(j):
                plsc.store_scatter(stg, [j * 17 + lane], edge_partial(j))

            for g in range((C + L - 1) // L):
                gl = (g * L + lane) * 17
                vs = [plsc.load_gather(stg, [gl + l]) for l in range(L)]
                while len(vs) > 1:
                    vs = [a + b for a, b in zip(vs[::2], vs[1::2])]
                ob[pl.ds(k * C + g * L, L)] = vs[0]

        fire_idx(0, 0)
        fire_idx(1, 1)
        fire_idx(2, 2)
        drain_idx(0)
        fire_rows(0, 0)

        def step(k, p, pi, last=False):
            if not last:
                drain_idx((pi + 1) % 4)
                fire_rows(1 - p, (pi + 1) % 4)
            drain_rows(p)

            @pl.when(k + 3 < nk)
            def _():
                fire_idx(k + 3, (pi + 3) % 4)
            compute(k, p)

        @pl.loop(0, nk // 4)
        def _t(tt):
            k = tt * 4
            for u in range(4):
                step(k + u, u % 2, u)

        for u in range(nk % 4):
            k = nk - (nk % 4) + u
            step(k, k % 2, k % 4, last=(u == nk % 4 - 1))
        pltpu.sync_copy(ob.at[pl.ds(0, per_w)],
                        out_hbm.at[pl.ds(base0, per_w)])

    return decode(x, src, dst)


def kernel(x, edge_index, edge_features, batch_size,
           W1, a_src1, a_dst1, We1, ae1, b1,
           W2, a_src2, a_dst2, We2, ae2, b2):
    n, d = x.shape
    xb = x.astype(jnp.bfloat16).reshape(n, d // 2, 2)
    xi = lax.bitcast_convert_type(xb, jnp.int32)
    xdup = jnp.broadcast_to(xi[:, None, :], (n, 2, d // 2)).reshape(n, d)
    res_half = _decode(xdup, edge_index[0], edge_index[1])
    return res_half.reshape((1000, -1))


# speedup vs baseline: 1.1161x; 1.1161x over previous
import functools

import jax
import jax.numpy as jnp
from jax import lax
from jax.experimental import pallas as pl
from jax.experimental.pallas import tpu as pltpu
from jax.experimental.pallas import tpu_sc as plsc

NC = 2
NS = 16
NW = NC * NS
L = 16
C = 40


def _decode(x, src, dst):
    n, d = x.shape
    e = src.shape[0]
    half = e // 2
    per_w = half // NW
    nk = per_w // C

    mesh = plsc.VectorSubcoreMesh(
        core_axis_name="c", subcore_axis_name="s",
        num_cores=NC, num_subcores=NS)

    @functools.partial(
        pl.kernel,
        out_type=jax.ShapeDtypeStruct((half,), jnp.float32),
        mesh=mesh,
        scratch_types=[
            [[pltpu.VMEM((C,), jnp.int32) for _ in range(4)]
             for _ in range(4)],
            [pltpu.VMEM((C, d), jnp.float32) for _ in range(4)],
            [pltpu.VMEM((C, d), jnp.float32) for _ in range(4)],
            pltpu.VMEM((per_w + 2 * L,), jnp.float32),
            pltpu.VMEM((3 * L * 17,), jnp.float32),
            pltpu.VMEM_SHARED((n, d), jnp.float32),
            [pltpu.SemaphoreType.DMA for _ in range(4)],
            pltpu.SemaphoreType.DMA,
            pltpu.SemaphoreType.DMA,
        ],
        compiler_params=pltpu.CompilerParams(needs_layout_passes=False),
    )
    def decode(x_hbm, src_hbm, dst_hbm, out_hbm,
               ias, set0, set1, ob, stg, xs, sis, sr0, sr1):
        wid = lax.axis_index("s") * NC + lax.axis_index("c")
        sid = lax.axis_index("s")
        base0 = wid * per_w
        lane = lax.iota(jnp.int32, L)
        sets = (set0, set1)
        srs = (sr0, sr1)

        rps = (n // NS) // 8 * 8
        off = pl.multiple_of(sid * rps, 8)
        pltpu.sync_copy(x_hbm.at[pl.ds(off, rps)], xs.at[pl.ds(off, rps)])
        tail = n - rps * NS
        if tail:
            @pl.when(sid == 0)
            def _tail():
                pltpu.sync_copy(x_hbm.at[pl.ds(rps * NS, tail)],
                                xs.at[pl.ds(rps * NS, tail)])
        plsc.subcore_barrier()

        def fire_idx(k, p):
            o = base0 + k * C
            pltpu.async_copy(src_hbm.at[pl.ds(o, C)], ias[p][0], sis[p])
            pltpu.async_copy(dst_hbm.at[pl.ds(o, C)], ias[p][1], sis[p])
            pltpu.async_copy(src_hbm.at[pl.ds(o + half, C)], ias[p][2],
                             sis[p])
            pltpu.async_copy(dst_hbm.at[pl.ds(o + half, C)], ias[p][3],
                             sis[p])

        def drain_idx(p):
            for q in range(4):
                pltpu.make_async_copy(src_hbm.at[pl.ds(0, C)], ias[p][q],
                                      sis[p]).wait()

        def fire_rows(p, pi):
            for q in range(4):
                pltpu.async_copy(xs.at[ias[pi][q]], sets[p][q], srs[p])

        def drain_rows(p):
            for q in range(4):
                pltpu.make_async_copy(xs.at[ias[0][q]], sets[p][q],
                                      srs[p]).wait()

        def compute(k, p):
            rs1, rd1, rs2, rd2 = sets[p]

            def edge_partial(j):
                ps = []
                for c0 in range(d // L):
                    sl = pl.ds(c0 * L, L)
                    ps.append(rs1[j, sl] * rd1[j, sl])
                    ps.append(rs2[j, sl] * rd2[j, sl])
                while len(ps) > 1:
                    ps = [a + b for a, b in zip(ps[::2], ps[1::2])]
                return ps[0]

            @plsc.parallel_loop(0, C)
            def _j1(j):
                plsc.store_scatter(stg, [j * 17 + lane], edge_partial(j))

            for g in range((C + L - 1) // L):
                gl = (g * L + lane) * 17
                vs = [plsc.load_gather(stg, [gl + l]) for l in range(L)]
                while len(vs) > 1:
                    vs = [a + b for a, b in zip(vs[::2], vs[1::2])]
                ob[pl.ds(k * C + g * L, L)] = vs[0]

        fire_idx(0, 0)
        fire_idx(1, 1)
        fire_idx(2, 2)
        drain_idx(0)
        fire_rows(0, 0)

        def step(k, p, pi, last=False):
            if not last:
                drain_idx((pi + 1) % 4)
                fire_rows(1 - p, (pi + 1) % 4)
            drain_rows(p)

            @pl.when(k + 3 < nk)
            def _():
                fire_idx(k + 3, (pi + 3) % 4)
            compute(k, p)

        @pl.loop(0, nk // 4)
        def _t(tt):
            k = tt * 4
            for u in range(4):
                step(k + u, u % 2, u)

        for u in range(nk % 4):
            k = nk - (nk % 4) + u
            step(k, k % 2, k % 4, last=(u == nk % 4 - 1))
        pltpu.sync_copy(ob.at[pl.ds(0, per_w)],
                        out_hbm.at[pl.ds(base0, per_w)])

    return decode(x, src, dst)


def kernel(x, edge_index, edge_features, batch_size,
           W1, a_src1, a_dst1, We1, ae1, b1,
           W2, a_src2, a_dst2, We2, ae2, b2):
    res_half = _decode(x, edge_index[0], edge_index[1])
    return res_half.reshape((1000, -1))


# speedup vs baseline: 1.1202x; 1.0037x over previous
import functools

import jax
import jax.numpy as jnp
from jax import lax
from jax.experimental import pallas as pl
from jax.experimental.pallas import tpu as pltpu
from jax.experimental.pallas import tpu_sc as plsc

NC = 2
NS = 16
NW = NC * NS
L = 16
C = 40


def _decode(x, src, dst):
    n, d = x.shape
    e = src.shape[0]
    half = e // 2
    per_w = half // NW
    nk = per_w // C

    mesh = plsc.VectorSubcoreMesh(
        core_axis_name="c", subcore_axis_name="s",
        num_cores=NC, num_subcores=NS)

    @functools.partial(
        pl.kernel,
        out_type=jax.ShapeDtypeStruct((half,), jnp.float32),
        mesh=mesh,
        scratch_types=[
            [pltpu.VMEM((4 * C,), jnp.int32) for _ in range(4)],
            pltpu.VMEM((4 * C, d), jnp.float32),
            pltpu.VMEM((4 * C, d), jnp.float32),
            pltpu.VMEM((per_w + 2 * L,), jnp.float32),
            pltpu.VMEM((3 * L * 17,), jnp.float32),
            pltpu.VMEM_SHARED((n, d), jnp.float32),
            [pltpu.SemaphoreType.DMA for _ in range(4)],
            pltpu.SemaphoreType.DMA,
            pltpu.SemaphoreType.DMA,
        ],
        compiler_params=pltpu.CompilerParams(needs_layout_passes=False),
    )
    def decode(x_hbm, src_hbm, dst_hbm, out_hbm,
               ias, set0, set1, ob, stg, xs, sis, sr0, sr1):
        wid = lax.axis_index("s") * NC + lax.axis_index("c")
        sid = lax.axis_index("s")
        base0 = wid * per_w
        lane = lax.iota(jnp.int32, L)
        sets = (set0, set1)
        srs = (sr0, sr1)

        rps = (n // NS) // 8 * 8
        off = pl.multiple_of(sid * rps, 8)
        pltpu.sync_copy(x_hbm.at[pl.ds(off, rps)], xs.at[pl.ds(off, rps)])
        tail = n - rps * NS
        if tail:
            @pl.when(sid == 0)
            def _tail():
                pltpu.sync_copy(x_hbm.at[pl.ds(rps * NS, tail)],
                                xs.at[pl.ds(rps * NS, tail)])
        plsc.subcore_barrier()

        def fire_idx(k, p):
            o = base0 + k * C
            srcs = (src_hbm.at[pl.ds(o, C)],
                    dst_hbm.at[pl.ds(o, C)],
                    src_hbm.at[pl.ds(o + half, C)],
                    dst_hbm.at[pl.ds(o + half, C)])
            for q in range(4):
                pltpu.async_copy(srcs[q], ias[p].at[pl.ds(q * C, C)],
                                 sis[p])

        def drain_idx(p):
            pltpu.make_async_copy(src_hbm.at[pl.ds(0, 4 * C)], ias[p],
                                  sis[p]).wait()

        def fire_rows(p, pi):
            for q in range(4):
                pltpu.async_copy(xs.at[ias[pi].at[pl.ds(q * C, C)]],
                                 sets[p].at[pl.ds(q * C, C)], srs[p])

        def drain_rows(p):
            pltpu.make_async_copy(x_hbm.at[pl.ds(0, 4 * C)], sets[p],
                                  srs[p]).wait()

        def compute(k, p):
            st = sets[p]

            def edge_partial(j):
                ps = []
                for c0 in range(d // L):
                    sl = pl.ds(c0 * L, L)
                    ps.append(st[j, sl] * st[C + j, sl])
                    ps.append(st[2 * C + j, sl] * st[3 * C + j, sl])
                while len(ps) > 1:
                    ps = [a + b for a, b in zip(ps[::2], ps[1::2])]
                return ps[0]

            @plsc.parallel_loop(0, C)
            def _j1(j):
                plsc.store_scatter(stg, [j * 17 + lane], edge_partial(j))

            for g in range((C + L - 1) // L):
                gl = (g * L + lane) * 17
                vs = [plsc.load_gather(stg, [gl + l]) for l in range(L)]
                while len(vs) > 1:
                    vs = [a + b for a, b in zip(vs[::2], vs[1::2])]
                ob[pl.ds(k * C + g * L, L)] = vs[0]

        fire_idx(0, 0)
        fire_idx(1, 1)
        fire_idx(2, 2)
        drain_idx(0)
        fire_rows(0, 0)

        def step(k, p, pi, last=False):
            if not last:
                drain_idx((pi + 1) % 4)
                fire_rows(1 - p, (pi + 1) % 4)
            drain_rows(p)

            @pl.when(k + 3 < nk)
            def _():
                fire_idx(k + 3, (pi + 3) % 4)
            compute(k, p)

        @pl.loop(0, nk // 4)
        def _t(tt):
            k = tt * 4
            for u in range(4):
                step(k + u, u % 2, u)

        for u in range(nk % 4):
            k = nk - (nk % 4) + u
            step(k, k % 2, k % 4, last=(u == nk % 4 - 1))
        pltpu.sync_copy(ob.at[pl.ds(0, per_w)],
                        out_hbm.at[pl.ds(base0, per_w)])

    return decode(x, src, dst)


def kernel(x, edge_index, edge_features, batch_size,
           W1, a_src1, a_dst1, We1, ae1, b1,
           W2, a_src2, a_dst2, We2, ae2, b2):
    res_half = _decode(x, edge_index[0], edge_index[1])
    return res_half.reshape((1000, -1))
